# Initial kernel scaffold; baseline (speedup 1.0000x reference)
import jax
import jax.numpy as jnp
from jax.experimental import pallas as pl


def kernel(size_scores, size_class_label, size_residual_label, size_residuals_normalized, object_assignment, objectness_label, mean_size_arr):
    raise NotImplementedError("write your pallas kernel here")


# speedup vs baseline: 11.5265x; 11.5265x over previous
import jax
import jax.numpy as jnp
from jax import lax
from jax.experimental import pallas as pl
from jax.experimental.pallas import tpu as pltpu

B, K, K2, NS = 32, 1024, 256, 18


def _huber(x, delta=1.0):
    ax = jnp.abs(x)
    return jnp.where(ax <= delta, 0.5 * x * x, delta * (ax - 0.5 * delta))


def _tc_body(scores_ref, res54_ref, cls_ref, rlab_ref, oa_ref, obj_ref, msa_ref,
             out_ref):
    b = pl.program_id(0)

    scores = scores_ref[0]
    oa = oa_ref[0, 0]
    labels = cls_ref[0, 0]
    w = obj_ref[0, 0]

    eq = oa[:, None] == lax.broadcasted_iota(jnp.int32, (K, K2), 1)
    cls = jnp.sum(jnp.where(eq, labels[None, :], 0), axis=1)

    m = jnp.max(scores, axis=-1)
    logZ = m + jnp.log(jnp.sum(jnp.exp(scores - m[:, None]), axis=-1))
    onehot = lax.broadcasted_iota(jnp.int32, (K, NS), 1) == cls[:, None]
    picked = jnp.sum(jnp.where(onehot, scores, 0.0), axis=1)
    ce_part = jnp.sum((logZ - picked) * w)
    w_part = jnp.sum(w)

    res54 = res54_ref[0]
    j = lax.broadcasted_iota(jnp.int32, (K, NS * 3), 1)
    n_idx = j // 3
    c_idx = j - 3 * n_idx
    oh54 = n_idx == cls[:, None]
    msa54 = msa_ref[0, 0]

    hub = jnp.zeros((K,), jnp.float32)
    for c in range(3):
        sel = oh54 & (c_idx == c)
        pred_c = jnp.sum(jnp.where(sel, res54, 0.0), axis=1)
        mean_c = jnp.sum(jnp.where(sel, msa54[None, :], 0.0), axis=1)
        rl_c = jnp.sum(jnp.where(eq, rlab_ref[0, c][None, :], 0.0), axis=1)
        diff = pred_c - rl_c / (mean_c + 1e-6)
        hub = hub + _huber(diff)
    res_part = jnp.sum(hub * (1.0 / 3.0) * w)

    lane = lax.broadcasted_iota(jnp.int32, (1, 128), 1)
    partial = (jnp.where(lane == 0, ce_part, 0.0)
               + jnp.where(lane == 1, res_part, 0.0)
               + jnp.where(lane == 2, w_part, 0.0))

    @pl.when(b == 0)
    def _():
        out_ref[...] = jnp.zeros_like(out_ref)

    out_ref[...] += partial

    @pl.when(b == B - 1)
    def _():
        acc = out_ref[...]
        denom = jnp.sum(jnp.where(lane == 2, acc, 0.0)) + 1e-6
        ce_sum = jnp.sum(jnp.where(lane == 0, acc, 0.0))
        res_sum = jnp.sum(jnp.where(lane == 1, acc, 0.0))
        out_ref[...] = (jnp.where(lane == 0, ce_sum / denom, 0.0)
                        + jnp.where(lane == 1, res_sum / denom, 0.0))


def kernel(size_scores, size_class_label, size_residual_label,
           size_residuals_normalized, object_assignment, objectness_label,
           mean_size_arr):
    res54 = size_residuals_normalized.reshape(B, K, NS * 3)
    cls3 = size_class_label.reshape(B, 1, K2)
    rlab_t = jnp.transpose(size_residual_label, (0, 2, 1))
    oa3 = object_assignment.reshape(B, 1, K)
    obj3 = objectness_label.reshape(B, 1, K)
    msa = mean_size_arr.reshape(1, 1, NS * 3)

    out = pl.pallas_call(
        _tc_body,
        grid=(B,),
        in_specs=[
            pl.BlockSpec((1, K, NS), lambda b: (b, 0, 0)),
            pl.BlockSpec((1, K, NS * 3), lambda b: (b, 0, 0)),
            pl.BlockSpec((1, 1, K2), lambda b: (b, 0, 0)),
            pl.BlockSpec((1, 3, K2), lambda b: (b, 0, 0)),
            pl.BlockSpec((1, 1, K), lambda b: (b, 0, 0)),
            pl.BlockSpec((1, 1, K), lambda b: (b, 0, 0)),
            pl.BlockSpec((1, 1, NS * 3), lambda b: (0, 0, 0)),
        ],
        out_specs=pl.BlockSpec((1, 128), lambda b: (0, 0)),
        out_shape=jax.ShapeDtypeStruct((1, 128), jnp.float32),
    )(size_scores, res54, cls3, rlab_t, oa3, obj3, msa)

    return out[0, 0], out[0, 1]
